# Initial kernel scaffold; baseline (speedup 1.0000x reference)
import jax
import jax.numpy as jnp
from jax.experimental import pallas as pl


def kernel(ids, table):
    raise NotImplementedError("write your pallas kernel here")


# speedup vs baseline: 1.4700x; 1.4700x over previous
import functools
import jax
import jax.numpy as jnp
from jax import lax
from jax.experimental import pallas as pl
from jax.experimental.pallas import tpu as pltpu
from jax.experimental.pallas import tpu_sc as plsc

DIM = 32
ROWS = 4096 * 200
NC, NS = 2, 16
NW = NC * NS
CHUNK = 1280
K = CHUNK // 128
NCHUNK = ROWS // (NW * CHUNK)

_mesh = plsc.VectorSubcoreMesh(core_axis_name="c", subcore_axis_name="s")


@functools.partial(
    pl.kernel,
    mesh=_mesh,
    compiler_params=pltpu.CompilerParams(use_tc_tiling_on_sc=False),
    out_type=jax.ShapeDtypeStruct((ROWS, DIM), jnp.float32),
    scratch_types=[
        pltpu.VMEM((K, 128), jnp.int32),
        pltpu.VMEM((CHUNK, DIM), jnp.float32),
        pltpu.SemaphoreType.DMA,
    ],
)
def _embed(ids_hbm, table_hbm, out_hbm, idx_v, rows_v, sem):
    wid = lax.axis_index("s") * NC + lax.axis_index("c")
    base = wid * (NCHUNK * CHUNK)

    def body(g, _):
        pltpu.sync_copy(ids_hbm.at[wid, g], idx_v)
        copies = [
            pltpu.async_copy(
                table_hbm.at[idx_v.at[j]],
                rows_v.at[pl.ds(j * 128, 128)],
                sem,
            )
            for j in range(K)
        ]
        for c in copies:
            c.wait()
        pltpu.sync_copy(rows_v, out_hbm.at[pl.ds(base + g * CHUNK, CHUNK)])
        return 0

    lax.fori_loop(0, NCHUNK, body, 0)


def kernel(ids, table):
    flat = ids.reshape(NW, NCHUNK, K, 128)
    out = _embed(flat, table)
    return out.reshape(ids.shape[0], ids.shape[1], DIM)


# speedup vs baseline: 1.4934x; 1.0159x over previous
import functools
import jax
import jax.numpy as jnp
from jax import lax
from jax.experimental import pallas as pl
from jax.experimental.pallas import tpu as pltpu
from jax.experimental.pallas import tpu_sc as plsc

DIM = 32
ROWS = 4096 * 200
NC, NS = 2, 16
NW = NC * NS
CHUNK = 1280
K = CHUNK // 128
NCHUNK = ROWS // (NW * CHUNK)
IDXROWS = NCHUNK * K

_mesh = plsc.VectorSubcoreMesh(core_axis_name="c", subcore_axis_name="s")


@functools.partial(
    pl.kernel,
    mesh=_mesh,
    compiler_params=pltpu.CompilerParams(use_tc_tiling_on_sc=False),
    out_type=jax.ShapeDtypeStruct((ROWS, DIM), jnp.float32),
    scratch_types=[
        pltpu.VMEM((IDXROWS, 128), jnp.int32),
        pltpu.VMEM((CHUNK, DIM), jnp.float32),
        pltpu.VMEM((CHUNK, DIM), jnp.float32),
        pltpu.SemaphoreType.DMA,
        pltpu.SemaphoreType.DMA,
        pltpu.SemaphoreType.DMA,
        pltpu.SemaphoreType.DMA,
    ],
)
def _embed(ids_hbm, table_hbm, out_hbm, idx_v, rows0, rows1, sg0, sg1, sw0, sw1):
    wid = lax.axis_index("s") * NC + lax.axis_index("c")
    base = wid * (NCHUNK * CHUNK)
    rows = (rows0, rows1)
    sg = (sg0, sg1)
    sw = (sw0, sw1)

    pltpu.sync_copy(ids_hbm.at[wid], idx_v)

    def fire(g, b):
        for j in range(K):
            pltpu.async_copy(
                table_hbm.at[idx_v.at[g * K + j]],
                rows[b].at[pl.ds(j * 128, 128)],
                sg[b],
            )

    def wait_gather(b):
        pltpu.make_async_copy(
            out_hbm.at[pl.ds(base, CHUNK)], rows[b], sg[b]
        ).wait()

    def write(g, b):
        pltpu.async_copy(rows[b], out_hbm.at[pl.ds(base + g * CHUNK, CHUNK)], sw[b])

    def wait_write(b):
        pltpu.make_async_copy(
            rows[b], out_hbm.at[pl.ds(base, CHUNK)], sw[b]
        ).wait()

    fire(0, 0)
    wait_gather(0)
    write(0, 0)
    fire(1, 1)

    def body(i, _):
        g = 2 * i + 1
        wait_gather(1)
        write(g, 1)
        wait_write(0)
        fire(g + 1, 0)
        wait_gather(0)
        write(g + 1, 0)
        wait_write(1)
        fire(g + 2, 1)
        return 0

    lax.fori_loop(0, (NCHUNK - 2) // 2, body, 0)

    wait_gather(1)
    write(NCHUNK - 1, 1)
    wait_write(0)
    wait_write(1)


def kernel(ids, table):
    flat = ids.reshape(NW, IDXROWS, 128)
    out = _embed(flat, table)
    return out.reshape(ids.shape[0], ids.shape[1], DIM)


# speedup vs baseline: 1.4940x; 1.0005x over previous
import functools
import jax
import jax.numpy as jnp
from jax import lax
from jax.experimental import pallas as pl
from jax.experimental.pallas import tpu as pltpu
from jax.experimental.pallas import tpu_sc as plsc

DIM = 32
B = 4096
T = 200
NC, NS = 2, 16
NW = NC * NS
PB = B // NW
CB = 8
NCHUNK = PB // CB
SPLITS = ((0, 104), (104, 96))

_mesh = plsc.VectorSubcoreMesh(core_axis_name="c", subcore_axis_name="s")


@functools.partial(
    pl.kernel,
    mesh=_mesh,
    compiler_params=pltpu.CompilerParams(use_tc_tiling_on_sc=False),
    out_type=jax.ShapeDtypeStruct((B, T, DIM), jnp.float32),
    scratch_types=[
        pltpu.VMEM((PB, T), jnp.int32),
        pltpu.VMEM((CB, T, DIM), jnp.float32),
        pltpu.VMEM((CB, T, DIM), jnp.float32),
        pltpu.SemaphoreType.DMA,
        pltpu.SemaphoreType.DMA,
        pltpu.SemaphoreType.DMA,
        pltpu.SemaphoreType.DMA,
    ],
)
def _embed(ids_hbm, table_hbm, out_hbm, idx_v, rows0, rows1, sg0, sg1, sw0, sw1):
    wid = lax.axis_index("s") * NC + lax.axis_index("c")
    rbase = wid * PB
    rows = (rows0, rows1)
    sg = (sg0, sg1)
    sw = (sw0, sw1)

    pltpu.sync_copy(ids_hbm.at[pl.ds(rbase, PB)], idx_v)

    def fire(g, b):
        for i in range(CB):
            for off, n in SPLITS:
                pltpu.async_copy(
                    table_hbm.at[idx_v.at[g * CB + i, pl.ds(off, n)]],
                    rows[b].at[i, pl.ds(off, n)],
                    sg[b],
                )

    def wait_gather(b):
        pltpu.make_async_copy(
            out_hbm.at[pl.ds(rbase, CB)], rows[b], sg[b]
        ).wait()

    def write(g, b):
        pltpu.async_copy(rows[b], out_hbm.at[pl.ds(rbase + g * CB, CB)], sw[b])

    def wait_write(b):
        pltpu.make_async_copy(
            rows[b], out_hbm.at[pl.ds(rbase, CB)], sw[b]
        ).wait()

    fire(0, 0)
    wait_gather(0)
    write(0, 0)
    fire(1, 1)

    def body(i, _):
        g = 2 * i + 1
        wait_gather(1)
        write(g, 1)
        wait_write(0)
        fire(g + 1, 0)
        wait_gather(0)
        write(g + 1, 0)
        wait_write(1)
        fire(g + 2, 1)
        return 0

    lax.fori_loop(0, (NCHUNK - 2) // 2, body, 0)

    wait_gather(1)
    write(NCHUNK - 1, 1)
    wait_write(0)
    wait_write(1)


def kernel(ids, table):
    return _embed(ids, table)
